# Initial kernel scaffold; baseline (speedup 1.0000x reference)
import functools

import jax
import jax.numpy as jnp
from jax import lax
from jax.experimental import pallas as pl
from jax.experimental.pallas import tpu as pltpu
from jax.experimental.pallas import tpu_sc as plsc

N = 16384
D = 256
L = 16
NW = 32
TR = 128
NTILES = N // TR
TPW = NTILES // NW
W = 128


def _sc_kernel(x_hbm, cols_hbm, vals_hbm, rows_hbm, bounds_hbm, out_hbm,
               acc, g, colbuf, valbuf, rowbuf, boundsbuf, sem):
    wid = lax.axis_index("c") * 16 + lax.axis_index("s")

    pltpu.sync_copy(bounds_hbm, boundsbuf)

    lane_iota = lax.iota(jnp.int32, L)
    zeros16 = jnp.zeros((L,), jnp.float32)

    for i in range(TPW):
        tile = wid * TPW + i
        tile_base = tile * TR
        e_start = boundsbuf[tile]
        e_end = boundsbuf[tile + 1]
        a_start = (e_start // 8) * 8
        nwin = (e_end - a_start + (W - 1)) // W

        @pl.loop(0, TR)
        def _(r):
            for c in range(D // L):
                acc[r, pl.ds(c * L, L)] = zeros16

        es_splat = jnp.full((L,), e_start, jnp.int32)
        ee_splat = jnp.full((L,), e_end, jnp.int32)
        tb_splat = jnp.full((L,), tile_base, jnp.int32)

        def window_body(wi, carry):
            e_base = a_start + wi * W
            pltpu.sync_copy(cols_hbm.at[pl.ds(e_base, W)], colbuf)
            pltpu.sync_copy(vals_hbm.at[pl.ds(e_base, W)], valbuf)
            pltpu.sync_copy(rows_hbm.at[pl.ds(e_base, W)], rowbuf)
            pltpu.async_copy(x_hbm.at[colbuf], g, sem).wait()
            eb_splat = jnp.full((L,), e_base, jnp.int32)

            @pl.loop(0, W)
            def _(e):
                e_splat = jnp.full((L,), e, jnp.int32)
                eg = eb_splat + e_splat
                m = jnp.logical_and(eg >= es_splat, eg < ee_splat)
                rv = plsc.load_gather(rowbuf, [e_splat])
                lr = rv - tb_splat
                vv = plsc.load_gather(valbuf, [e_splat])
                nv = -vv
                for c in range(D // L):
                    gch = g[e, pl.ds(c * L, L)]
                    t = gch * nv
                    plsc.addupdate_scatter(
                        acc, [lr, lane_iota + (c * L)], t, mask=m)

            return carry

        lax.fori_loop(0, nwin, window_body, 0)

        pltpu.sync_copy(acc, out_hbm.at[pl.ds(tile_base, TR)])


def kernel(t, x, L_rows, L_cols, L_vals):
    del t
    tile_starts = jnp.arange(0, N + 1, TR, dtype=jnp.int32)
    bounds = jnp.searchsorted(L_rows, tile_starts, side="left").astype(jnp.int32)
    bounds = jnp.concatenate([bounds, jnp.zeros((7,), jnp.int32)])
    pad_i = jnp.zeros((W,), jnp.int32)
    pad_f = jnp.zeros((W,), jnp.float32)
    cols_p = jnp.concatenate([L_cols, pad_i])
    rows_p = jnp.concatenate([L_rows, pad_i])
    vals_p = jnp.concatenate([L_vals, pad_f])

    mesh = plsc.VectorSubcoreMesh(core_axis_name="c", subcore_axis_name="s")
    run = pl.kernel(
        _sc_kernel,
        out_type=jax.ShapeDtypeStruct((N, D), jnp.float32),
        mesh=mesh,
        scratch_types=[
            pltpu.VMEM((TR, D), jnp.float32),
            pltpu.VMEM((W, D), jnp.float32),
            pltpu.VMEM((W,), jnp.int32),
            pltpu.VMEM((W,), jnp.float32),
            pltpu.VMEM((W,), jnp.int32),
            pltpu.VMEM((136,), jnp.int32),
            pltpu.SemaphoreType.DMA,
        ],
    )
    return run(x, cols_p, vals_p, rows_p, bounds)


# speedup vs baseline: 2.4202x; 2.4202x over previous
import dataclasses
import functools

import jax
import jax.numpy as jnp
from jax import lax
from jax.experimental import pallas as pl
from jax.experimental.pallas import tpu as pltpu
from jax.experimental.pallas import tpu_sc as plsc

N = 16384
D = 256
L = 16
NW = 32
TR = 128
NTILES = N // TR
TPW = NTILES // NW
W = 128


def _sc_kernel(x_hbm, cols_hbm, vals_hbm, rows_hbm, bounds_hbm, out_hbm,
               acc, g, colbuf, valbuf, rowbuf, boundsbuf, sem):
    wid = lax.axis_index("c") * 16 + lax.axis_index("s")

    pltpu.sync_copy(bounds_hbm, boundsbuf)

    lane_iota = lax.iota(jnp.int32, L)
    zeros16 = jnp.zeros((L,), jnp.float32)

    for i in range(TPW):
        tile = wid * TPW + i
        tile_base = tile * TR
        bv = boundsbuf[pl.ds(tile, L)]
        e_start = bv[0]
        e_end = bv[1]
        a_start = (e_start // 8) * 8
        nwin = (e_end - a_start + (W - 1)) // W

        @pl.loop(0, TR)
        def _(r):
            for c in range(D // L):
                acc[r, pl.ds(c * L, L)] = zeros16

        es_splat = jnp.full((L,), e_start, jnp.int32)
        ee_splat = jnp.full((L,), e_end, jnp.int32)
        tb_splat = jnp.full((L,), tile_base, jnp.int32)

        def window_body(wi, carry):
            e_base = a_start + wi * W
            pltpu.sync_copy(cols_hbm.at[pl.ds(e_base, W)], colbuf)
            pltpu.sync_copy(vals_hbm.at[pl.ds(e_base, W)], valbuf)
            pltpu.sync_copy(rows_hbm.at[pl.ds(e_base, W)], rowbuf)
            pltpu.async_copy(x_hbm.at[colbuf], g, sem).wait()
            eb_splat = jnp.full((L,), e_base, jnp.int32)

            @pl.loop(0, W)
            def _(e):
                e_splat = jnp.full((L,), e, jnp.int32)
                eg = eb_splat + e_splat
                m = jnp.logical_and(eg >= es_splat, eg < ee_splat)
                rv = plsc.load_gather(rowbuf, [e_splat])
                lr = rv - tb_splat
                vv = plsc.load_gather(valbuf, [e_splat])
                nv = -vv
                for c in range(D // L):
                    gch = g[e, pl.ds(c * L, L)]
                    t = gch * nv
                    plsc.addupdate_scatter(
                        acc, [lr, lane_iota + (c * L)], t, mask=m)

            return carry

        lax.fori_loop(0, nwin, window_body, 0)

        pltpu.sync_copy(acc, out_hbm.at[pl.ds(tile_base, TR)])


def kernel(t, x, L_rows, L_cols, L_vals):
    del t
    tile_starts = jnp.arange(0, N + 1, TR, dtype=jnp.int32)
    bounds = jnp.searchsorted(L_rows, tile_starts, side="left").astype(jnp.int32)
    bounds = jnp.concatenate([bounds, jnp.zeros((7,), jnp.int32)])
    pad_i = jnp.zeros((W,), jnp.int32)
    pad_f = jnp.zeros((W,), jnp.float32)
    cols_p = jnp.concatenate([L_cols, pad_i])
    rows_p = jnp.concatenate([L_rows, pad_i])
    vals_p = jnp.concatenate([L_vals, pad_f])

    mesh = plsc.VectorSubcoreMesh(core_axis_name="c", subcore_axis_name="s")
    cp = pltpu.CompilerParams()
    if "needs_layout_passes" in pltpu.CompilerParams.__dataclass_fields__:
        cp = dataclasses.replace(cp, needs_layout_passes=False)
    run = pl.kernel(
        _sc_kernel,
        out_type=jax.ShapeDtypeStruct((N, D), jnp.float32),
        mesh=mesh,
        scratch_types=[
            pltpu.VMEM((TR, D), jnp.float32),
            pltpu.VMEM((W, D), jnp.float32),
            pltpu.VMEM((W,), jnp.int32),
            pltpu.VMEM((W,), jnp.float32),
            pltpu.VMEM((W,), jnp.int32),
            pltpu.VMEM((136,), jnp.int32),
            pltpu.SemaphoreType.DMA,
        ],
        compiler_params=cp,
    )
    return run(x, cols_p, vals_p, rows_p, bounds)
